# baseline (speedup 1.0000x reference)
import math

import jax
import jax.numpy as jnp
import numpy as np
from jax.experimental import pallas as pl
from jax.experimental.pallas import tpu as pltpu

_D = 2048
_ROWS_PER_BLK = 256


def _make_wph():
    div_term = np.exp(
        np.arange(0, _D, 2, dtype=np.float32) * -(math.log(10000.0) / _D))
    w = np.repeat(div_term, 2)
    ph = np.tile(np.array([0.0, np.pi / 2], dtype=np.float32), _D // 2)
    return jnp.asarray(w[None, :]), jnp.asarray(ph[None, :])


def _sin_body(pos_ref, w_ref, ph_ref, out_ref):
    p = pos_ref[...].astype(jnp.float32)
    arg = p * w_ref[...] + ph_ref[...]
    out_ref[...] = jnp.sin(arg)


def kernel(positions, pe):
    b, s = positions.shape
    d = pe.shape[1]
    n = b * s
    flat = positions.reshape(n, 1)
    w, ph = _make_wph()
    grid = n // _ROWS_PER_BLK
    out = pl.pallas_call(
        _sin_body,
        grid=(grid,),
        in_specs=[
            pl.BlockSpec((_ROWS_PER_BLK, 1), lambda i: (i, 0)),
            pl.BlockSpec((1, d), lambda i: (0, 0)),
            pl.BlockSpec((1, d), lambda i: (0, 0)),
        ],
        out_specs=pl.BlockSpec((_ROWS_PER_BLK, d), lambda i: (i, 0)),
        out_shape=jax.ShapeDtypeStruct((n, d), jnp.float32),
    )(flat, w, ph)
    return out.reshape(b, s, d)
